# baseline (speedup 1.0000x reference)
import jax
import jax.numpy as jnp
from jax.experimental import pallas as pl

POSITIONS = 200
GROUP = 32
LANES = GROUP * POSITIONS
BLOCK_ROWS = 64


def _onehot_block(x_ref, out_ref):
    x = x_ref[...].astype(jnp.float32)

    m_i = jax.lax.broadcasted_iota(jnp.int32, (GROUP, LANES), 0)
    c_i = jax.lax.broadcasted_iota(jnp.int32, (GROUP, LANES), 1)
    sel = ((c_i >= POSITIONS * m_i) & (c_i < POSITIONS * m_i + POSITIONS))
    s = sel.astype(jnp.float32)
    j_map = jnp.sum(jnp.where(sel, c_i - POSITIONS * m_i, 0), axis=0)
    j_map = j_map.astype(jnp.float32)

    x_exp = jax.lax.dot(x, s, preferred_element_type=jnp.float32,
                        precision=jax.lax.Precision.HIGHEST)
    out_ref[...] = jnp.where(x_exp == j_map[None, :], 1.0, 0.0)


def kernel(inputs):
    n = inputs.shape[0]
    rows = n // GROUP
    grid = rows // BLOCK_ROWS
    x2d = inputs.reshape(rows, GROUP)
    out = pl.pallas_call(
        _onehot_block,
        grid=(grid,),
        in_specs=[pl.BlockSpec((BLOCK_ROWS, GROUP), lambda i: (i, 0))],
        out_specs=pl.BlockSpec((BLOCK_ROWS, LANES), lambda i: (i, 0)),
        out_shape=jax.ShapeDtypeStruct((rows, LANES), jnp.float32),
    )(x2d)
    return out.reshape(n, POSITIONS)


# speedup vs baseline: 1.0813x; 1.0813x over previous
import jax
import jax.numpy as jnp
from jax.experimental import pallas as pl

POSITIONS = 200
GROUP = 32
LANES = GROUP * POSITIONS
BLOCK_ROWS = 64


def _onehot_block(x_ref, out_ref):
    x = x_ref[...].astype(jnp.bfloat16)

    m_i = jax.lax.broadcasted_iota(jnp.int32, (GROUP, LANES), 0)
    c_i = jax.lax.broadcasted_iota(jnp.int32, (GROUP, LANES), 1)
    sel = ((c_i >= POSITIONS * m_i) & (c_i < POSITIONS * m_i + POSITIONS))
    s = sel.astype(jnp.bfloat16)
    j_map = jnp.sum(jnp.where(sel, c_i - POSITIONS * m_i, 0), axis=0)
    j_map = j_map.astype(jnp.float32)

    x_exp = jax.lax.dot(x, s, preferred_element_type=jnp.float32)
    out_ref[...] = jnp.where(x_exp == j_map[None, :], 1.0, 0.0)


def kernel(inputs):
    n = inputs.shape[0]
    rows = n // GROUP
    grid = rows // BLOCK_ROWS
    x2d = inputs.reshape(rows, GROUP)
    out = pl.pallas_call(
        _onehot_block,
        grid=(grid,),
        in_specs=[pl.BlockSpec((BLOCK_ROWS, GROUP), lambda i: (i, 0))],
        out_specs=pl.BlockSpec((BLOCK_ROWS, LANES), lambda i: (i, 0)),
        out_shape=jax.ShapeDtypeStruct((rows, LANES), jnp.float32),
    )(x2d)
    return out.reshape(n, POSITIONS)


# speedup vs baseline: 2.1119x; 1.9531x over previous
import jax
import jax.numpy as jnp
from jax.experimental import pallas as pl

POSITIONS = 200
BLOCK_ROWS = 512


def _onehot_block(in_ref, out_ref):
    vals = in_ref[0, 0, :]
    cols = jax.lax.broadcasted_iota(jnp.int32, (BLOCK_ROWS, POSITIONS), 1)
    out_ref[...] = (vals[:, None] == cols).astype(jnp.float32)


def kernel(inputs):
    n = inputs.shape[0]
    grid = n // BLOCK_ROWS
    inputs3 = inputs.reshape(grid, 1, BLOCK_ROWS)
    return pl.pallas_call(
        _onehot_block,
        grid=(grid,),
        in_specs=[pl.BlockSpec((1, 1, BLOCK_ROWS), lambda i: (i, 0, 0))],
        out_specs=pl.BlockSpec((BLOCK_ROWS, POSITIONS), lambda i: (i, 0)),
        out_shape=jax.ShapeDtypeStruct((n, POSITIONS), jnp.float32),
    )(inputs3)


# speedup vs baseline: 3.1533x; 1.4931x over previous
import jax
import jax.numpy as jnp
from jax.experimental import pallas as pl

POSITIONS = 200
BLOCK_ROWS = 8192


def _onehot_block(in_ref, out_ref):
    vals = in_ref[0, 0, :]
    cols = jax.lax.broadcasted_iota(jnp.int32, (BLOCK_ROWS, POSITIONS), 1)
    out_ref[...] = (vals[:, None] == cols).astype(jnp.float32)


def kernel(inputs):
    n = inputs.shape[0]
    grid = n // BLOCK_ROWS
    inputs3 = inputs.reshape(grid, 1, BLOCK_ROWS)
    return pl.pallas_call(
        _onehot_block,
        grid=(grid,),
        in_specs=[pl.BlockSpec((1, 1, BLOCK_ROWS), lambda i: (i, 0, 0))],
        out_specs=pl.BlockSpec((BLOCK_ROWS, POSITIONS), lambda i: (i, 0)),
        out_shape=jax.ShapeDtypeStruct((n, POSITIONS), jnp.float32),
    )(inputs3)


# speedup vs baseline: 3.2238x; 1.0223x over previous
import jax
import jax.numpy as jnp
from jax.experimental import pallas as pl
from jax.experimental.pallas import tpu as pltpu

POSITIONS = 200
CHUNK = 2048
NCHUNK = 8
NBUF = 8


def _onehot_kernel(in_ref, out_ref, scratch, sems):
    def body(i, _):
        slot = jax.lax.rem(i, NBUF)
        vals = in_ref[i, 0, :]
        cols = jax.lax.broadcasted_iota(jnp.int32, (CHUNK, POSITIONS), 1)
        scratch[slot] = (vals[:, None] == cols).astype(jnp.float32)
        pltpu.make_async_copy(
            scratch.at[slot],
            out_ref.at[pl.ds(i * CHUNK, CHUNK), :],
            sems.at[slot],
        ).start()
        return 0

    jax.lax.fori_loop(0, NCHUNK, body, 0)

    def drain(i, _):
        slot = jax.lax.rem(i, NBUF)
        pltpu.make_async_copy(
            scratch.at[slot],
            out_ref.at[pl.ds(i * CHUNK, CHUNK), :],
            sems.at[slot],
        ).wait()
        return 0

    jax.lax.fori_loop(0, NCHUNK, drain, 0)


def kernel(inputs):
    n = inputs.shape[0]
    inputs3 = inputs.reshape(NCHUNK, 1, CHUNK)
    return pl.pallas_call(
        _onehot_kernel,
        in_specs=[pl.BlockSpec(memory_space=pltpu.MemorySpace.VMEM)],
        out_specs=pl.BlockSpec(memory_space=pltpu.MemorySpace.HBM),
        out_shape=jax.ShapeDtypeStruct((n, POSITIONS), jnp.float32),
        scratch_shapes=[
            pltpu.VMEM((NBUF, CHUNK, POSITIONS), jnp.float32),
            pltpu.SemaphoreType.DMA((NBUF,)),
        ],
    )(inputs3)
